# baseline (speedup 1.0000x reference)
import functools

import jax
import jax.numpy as jnp
from jax import lax
from jax.experimental import pallas as pl
from jax.experimental.pallas import tpu as pltpu
from jax.experimental.pallas import tpu_sc as plsc

_MASK_RATIO = 0.75

_IC = 128
_JC = 128


def _rank_body(noise_row_ref, noise_col_ref, x_ref,
               rank_ref, mask_ref, xp_ref, *, L, len_keep):
    ic = pl.program_id(1)
    nj = L // _JC
    ki = noise_col_ref[0]
    ki_b = jnp.broadcast_to(ki, (_IC, _JC))

    def kj_at(jc):
        return noise_row_ref[0, :, pl.ds(jc * _JC, _JC)]

    def before(jc, acc):
        return acc + ((ki_b - kj_at(jc)) >> 31)

    def after(jc, acc):
        return acc - ((kj_at(jc) - ki_b) >> 31)

    U = 4

    def before_u(t, acc):
        for u in range(U):
            acc = before(t * U + u, acc)
        return acc

    def after_u(t, acc):
        for u in range(U):
            acc = after(ic + 1 + t * U + u, acc)
        return acc

    acc = jnp.zeros((_IC, _JC), jnp.int32)
    acc = lax.fori_loop(0, ic // U, before_u, acc)
    acc = lax.fori_loop((ic // U) * U, ic, before, acc)
    n_after = nj - ic - 1
    acc = lax.fori_loop(0, n_after // U, after_u, acc)
    acc = lax.fori_loop(ic + 1 + (n_after // U) * U, nj, after, acc)
    cki = ki_b * _JC + lax.broadcasted_iota(jnp.int32, (_IC, _JC), 0)
    ckj = kj_at(ic) * _JC + lax.broadcasted_iota(jnp.int32, (1, _JC), 1)
    acc = acc - ((ckj - cki) >> 31)

    rank = ic * _JC + jnp.sum(acc, axis=1, keepdims=True)
    rank_ref[0] = rank
    mask_ref[0] = (rank >= len_keep).astype(jnp.float32)

    v = x_ref[0, 0]
    xb = v.shape[0]
    xp_ref[0] = jnp.concatenate([v[: xb // 2], v[xb // 2:]], axis=1)


def _rank_and_repack(noise, x, interpret=False):
    N, L = noise.shape
    _, F, _, D = x.shape
    len_keep = int(L * (1 - _MASK_RATIO))
    steps = N * (L // _IC)
    xb = (N * F * L) // steps
    halves = L // xb
    keys = (noise * float(1 << 23)).astype(jnp.int32)
    body = functools.partial(_rank_body, L=L, len_keep=len_keep)

    def x_map(r, ic):
        s = r * (L // _IC) + ic
        return (s // (F * halves), (s // halves) % F, s % halves, 0)

    def xp_map(r, ic):
        return (r * (L // _IC) + ic, 0, 0)

    rank3, mask3, xp = pl.pallas_call(
        body,
        grid=(N, L // _IC),
        in_specs=[
            pl.BlockSpec((1, 1, L), lambda r, ic: (r, 0, 0)),
            pl.BlockSpec((1, _IC, 1), lambda r, ic: (r, ic, 0)),
            pl.BlockSpec((1, 1, xb, D), x_map),
        ],
        out_specs=[
            pl.BlockSpec((1, _IC, 1), lambda r, ic: (r, ic, 0)),
            pl.BlockSpec((1, _IC, 1), lambda r, ic: (r, ic, 0)),
            pl.BlockSpec((1, xb // 2, 2 * D), xp_map),
        ],
        out_shape=[
            jax.ShapeDtypeStruct((N, L, 1), jnp.int32),
            jax.ShapeDtypeStruct((N, L, 1), jnp.float32),
            jax.ShapeDtypeStruct((steps, xb // 2, 2 * D), jnp.float32),
        ],
        interpret=interpret,
    )(keys.reshape(N, 1, L), keys.reshape(N, L, 1), x)
    return (rank3.reshape(N, L), mask3.reshape(N, L),
            xp.reshape(N * F * L // 2, 2 * D))


_CH = 512


def _make_sc_gather(N, F, L, D, len_keep):
    n_rows_out = N * F * len_keep
    workers = 32
    rows_per_w = n_rows_out // workers
    f_per_w = rows_per_w // len_keep
    w_per_n = F // f_per_w
    chunks = rows_per_w // _CH
    mesh = plsc.VectorSubcoreMesh(core_axis_name="c", subcore_axis_name="s")

    @functools.partial(
        pl.kernel,
        mesh=mesh,
        out_type=[
            jax.ShapeDtypeStruct((n_rows_out, 2 * D), jnp.float32),
            jax.ShapeDtypeStruct((N, len_keep), jnp.int32),
        ],
        scratch_types=[
            pltpu.VMEM((L,), jnp.int32),
            pltpu.VMEM((len_keep,), jnp.int32),
            pltpu.VMEM((_CH,), jnp.int32),
            pltpu.VMEM((_CH, 2 * D), jnp.float32),
            pltpu.SemaphoreType.DMA,
        ],
        compiler_params=pltpu.CompilerParams(needs_layout_passes=False),
    )
    def sc_gather(rank_hbm, xp_hbm, y_hbm, keep_hbm,
                  rank_v, keep_v, idx_v, data_v, sem):
        c = lax.axis_index("c")
        s = lax.axis_index("s")
        w = s * 2 + c
        n = w // w_per_n
        f0 = (w % w_per_n) * f_per_w

        pltpu.sync_copy(rank_hbm.at[n], rank_v)

        def inv_body(i, carry):
            rk = rank_v[pl.ds(i * 16, 16)]
            vals = i * 16 + lax.iota(jnp.int32, 16)
            keepm = rk < len_keep
            idxc = jnp.where(keepm, rk, len_keep - 1)
            plsc.store_scatter(keep_v, [idxc], vals, mask=keepm)
            return carry

        lax.fori_loop(0, L // 16, inv_body, 0)

        @pl.when(w % w_per_n == 0)
        def _():
            pltpu.sync_copy(keep_v, keep_hbm.at[n])

        def ch_body(t, carry):
            f = f0 + t // (len_keep // _CH)
            koff = (t % (len_keep // _CH)) * _CH
            base = (n * F + f) * L

            def idx_body(q, carry2):
                kp = keep_v[pl.ds(koff + q * 16, 16)]
                g = base + kp
                idx_v[pl.ds(q * 16, 16)] = ((g >> 12) << 11) | (g & 2047)
                return carry2

            lax.fori_loop(0, _CH // 16, idx_body, 0)
            pltpu.async_copy(xp_hbm.at[idx_v], data_v, sem).wait()
            out_base = w * rows_per_w + t * _CH
            pltpu.sync_copy(data_v, y_hbm.at[pl.ds(out_base, _CH)])
            return carry

        lax.fori_loop(0, chunks, ch_body, 0)

    return sc_gather


def _compact(y2, keep, N, F, len_keep, D, interpret=False):
    par = ((keep >> 11) & 1).reshape(N, 1, len_keep, 1)

    def body(y_ref, p_ref, o_ref):
        h0 = y_ref[0, 0, :, :D]
        h1 = y_ref[0, 0, :, D:]
        o_ref[0, 0] = jnp.where(p_ref[0, 0] > 0, h1, h0)

    return pl.pallas_call(
        body,
        grid=(N, F),
        in_specs=[
            pl.BlockSpec((1, 1, len_keep, 2 * D), lambda n, f: (n, f, 0, 0)),
            pl.BlockSpec((1, 1, len_keep, 1), lambda n, f: (n, 0, 0, 0)),
        ],
        out_specs=pl.BlockSpec((1, 1, len_keep, D), lambda n, f: (n, f, 0, 0)),
        out_shape=jax.ShapeDtypeStruct((N, F, len_keep, D), jnp.float32),
        interpret=interpret,
    )(y2.reshape(N, F, len_keep, 2 * D), par)


def kernel(x):
    N, F, L, D = x.shape
    len_keep = int(L * (1 - _MASK_RATIO))
    noise = jax.random.uniform(jax.random.key(42), (N, L), dtype=x.dtype)
    rank, mask_row, xp = _rank_and_repack(noise, x)
    sc_gather = _make_sc_gather(N, F, L, D, len_keep)
    y2, keep = sc_gather(rank, xp)
    x_masked = _compact(y2, keep, N, F, len_keep, D)
    mask = jnp.broadcast_to(mask_row[:, None, :], (N, F, L))
    ids_restore = jnp.broadcast_to(rank[:, None, :], (N, F, L))
    return (x_masked, mask, ids_restore)


# speedup vs baseline: 2.0697x; 2.0697x over previous
import functools

import jax
import jax.numpy as jnp
from jax import lax
from jax.experimental import pallas as pl
from jax.experimental.pallas import tpu as pltpu
from jax.experimental.pallas import tpu_sc as plsc

_MASK_RATIO = 0.75

_IC = 128
_JC = 128


def _rank_body(noise_row_ref, noise_col_ref, rank_ref, mask_ref, *, L, len_keep):
    ic = pl.program_id(1)
    nj = L // _JC
    ki = noise_col_ref[0]
    ki_b = jnp.broadcast_to(ki, (_IC, _JC))

    def kj_at(jc):
        return noise_row_ref[0, :, pl.ds(jc * _JC, _JC)]

    def before(jc, acc):
        return acc + ((ki_b - kj_at(jc)) >> 31)

    def after(jc, acc):
        return acc - ((kj_at(jc) - ki_b) >> 31)

    U = 4

    def before_u(t, acc):
        for u in range(U):
            acc = before(t * U + u, acc)
        return acc

    def after_u(t, acc):
        for u in range(U):
            acc = after(ic + 1 + t * U + u, acc)
        return acc

    acc = jnp.zeros((_IC, _JC), jnp.int32)
    acc = lax.fori_loop(0, ic // U, before_u, acc)
    acc = lax.fori_loop((ic // U) * U, ic, before, acc)
    n_after = nj - ic - 1
    acc = lax.fori_loop(0, n_after // U, after_u, acc)
    acc = lax.fori_loop(ic + 1 + (n_after // U) * U, nj, after, acc)
    cki = ki_b * _JC + lax.broadcasted_iota(jnp.int32, (_IC, _JC), 0)
    ckj = kj_at(ic) * _JC + lax.broadcasted_iota(jnp.int32, (1, _JC), 1)
    acc = acc - ((ckj - cki) >> 31)

    rank = ic * _JC + jnp.sum(acc, axis=1, keepdims=True)
    rank_ref[0] = rank
    mask_ref[0] = (rank >= len_keep).astype(jnp.float32)


def _compute_rank(noise, interpret=False):
    N, L = noise.shape
    len_keep = int(L * (1 - _MASK_RATIO))
    keys = (noise * float(1 << 23)).astype(jnp.int32)
    body = functools.partial(_rank_body, L=L, len_keep=len_keep)
    rank3, mask3 = pl.pallas_call(
        body,
        grid=(N, L // _IC),
        in_specs=[
            pl.BlockSpec((1, 1, L), lambda r, ic: (r, 0, 0)),
            pl.BlockSpec((1, _IC, 1), lambda r, ic: (r, ic, 0)),
        ],
        out_specs=[
            pl.BlockSpec((1, _IC, 1), lambda r, ic: (r, ic, 0)),
            pl.BlockSpec((1, _IC, 1), lambda r, ic: (r, ic, 0)),
        ],
        out_shape=[
            jax.ShapeDtypeStruct((N, L, 1), jnp.int32),
            jax.ShapeDtypeStruct((N, L, 1), jnp.float32),
        ],
        interpret=interpret,
    )(keys.reshape(N, 1, L), keys.reshape(N, L, 1))
    return rank3.reshape(N, L), mask3.reshape(N, L)


_CR = 8


def _make_sc_gather(N, F, L, D, len_keep):
    n_rows = N * F * D
    workers = 32
    rows_per_w = n_rows // workers
    w_per_n = workers // N
    chunks = rows_per_w // _CR
    mesh = plsc.VectorSubcoreMesh(core_axis_name="c", subcore_axis_name="s")

    @functools.partial(
        pl.kernel,
        mesh=mesh,
        out_type=jax.ShapeDtypeStruct((n_rows, len_keep), jnp.float32),
        scratch_types=(
            [pltpu.VMEM((L,), jnp.int32), pltpu.VMEM((len_keep,), jnp.int32)]
            + [pltpu.VMEM((L,), jnp.float32) for _ in range(_CR)]
            + [pltpu.VMEM((len_keep,), jnp.float32) for _ in range(_CR)]
            + [pltpu.SemaphoreType.DMA]
        ),
        compiler_params=pltpu.CompilerParams(needs_layout_passes=False),
    )
    def sc_gather(rank_hbm, xt_hbm, yt_hbm, rank_v, keep_v, *rest):
        ins = rest[:_CR]
        outs = rest[_CR:2 * _CR]
        sem = rest[2 * _CR]
        c = lax.axis_index("c")
        s = lax.axis_index("s")
        w = s * 2 + c
        n = w // w_per_n
        row0 = w * rows_per_w

        pltpu.sync_copy(rank_hbm.at[n], rank_v)

        def inv_body(i, carry):
            rk = rank_v[pl.ds(i * 16, 16)]
            vals = i * 16 + lax.iota(jnp.int32, 16)
            keepm = rk < len_keep
            idxc = jnp.where(keepm, rk, len_keep - 1)
            plsc.store_scatter(keep_v, [idxc], vals, mask=keepm)
            return carry

        lax.fori_loop(0, L // 16, inv_body, 0)

        def ch_body(t, carry):
            rb = row0 + t * _CR
            handles = [
                pltpu.async_copy(xt_hbm.at[rb + rr], ins[rr], sem)
                for rr in range(_CR)
            ]
            for h in handles:
                h.wait()

            def q_body(q, carry2):
                idx = keep_v[pl.ds(q * 16, 16)]
                for rr in range(_CR):
                    outs[rr][pl.ds(q * 16, 16)] = plsc.load_gather(
                        ins[rr], [idx])
                return carry2

            lax.fori_loop(0, len_keep // 16, q_body, 0)
            for rr in range(_CR):
                pltpu.sync_copy(outs[rr], yt_hbm.at[rb + rr])
            return carry

        lax.fori_loop(0, chunks, ch_body, 0)

    return sc_gather


def kernel(x):
    N, F, L, D = x.shape
    len_keep = int(L * (1 - _MASK_RATIO))
    noise = jax.random.uniform(jax.random.key(42), (N, L), dtype=x.dtype)
    rank, mask_row = _compute_rank(noise)
    sc_gather = _make_sc_gather(N, F, L, D, len_keep)
    xt = jnp.transpose(x, (0, 1, 3, 2)).reshape(N * F * D, L)
    yt = sc_gather(rank, xt)
    x_masked = jnp.transpose(yt.reshape(N, F, D, len_keep), (0, 1, 3, 2))
    mask = jnp.broadcast_to(mask_row[:, None, :], (N, F, L))
    ids_restore = jnp.broadcast_to(rank[:, None, :], (N, F, L))
    return (x_masked, mask, ids_restore)


# speedup vs baseline: 2.6338x; 1.2726x over previous
import functools

import jax
import jax.numpy as jnp
from jax import lax
from jax.experimental import pallas as pl
from jax.experimental.pallas import tpu as pltpu
from jax.experimental.pallas import tpu_sc as plsc

_MASK_RATIO = 0.75

_IC = 128
_JC = 128


def _rank_body(noise_row_ref, noise_col_ref, rank_ref, mask_ref, *, L, len_keep):
    ic = pl.program_id(1)
    nj = L // _JC
    ki = noise_col_ref[0]
    ki_b = jnp.broadcast_to(ki, (_IC, _JC))

    def kj_at(jc):
        return noise_row_ref[0, :, pl.ds(jc * _JC, _JC)]

    def before(jc, acc):
        return acc + ((ki_b - kj_at(jc)) >> 31)

    def after(jc, acc):
        return acc - ((kj_at(jc) - ki_b) >> 31)

    U = 4

    def before_u(t, acc):
        for u in range(U):
            acc = before(t * U + u, acc)
        return acc

    def after_u(t, acc):
        for u in range(U):
            acc = after(ic + 1 + t * U + u, acc)
        return acc

    acc = jnp.zeros((_IC, _JC), jnp.int32)
    acc = lax.fori_loop(0, ic // U, before_u, acc)
    acc = lax.fori_loop((ic // U) * U, ic, before, acc)
    n_after = nj - ic - 1
    acc = lax.fori_loop(0, n_after // U, after_u, acc)
    acc = lax.fori_loop(ic + 1 + (n_after // U) * U, nj, after, acc)
    cki = ki_b * _JC + lax.broadcasted_iota(jnp.int32, (_IC, _JC), 0)
    ckj = kj_at(ic) * _JC + lax.broadcasted_iota(jnp.int32, (1, _JC), 1)
    acc = acc - ((ckj - cki) >> 31)

    rank = ic * _JC + jnp.sum(acc, axis=1, keepdims=True)
    rank_ref[0] = rank
    mask_ref[0] = (rank >= len_keep).astype(jnp.float32)


def _compute_rank(noise, interpret=False):
    N, L = noise.shape
    len_keep = int(L * (1 - _MASK_RATIO))
    keys = (noise * float(1 << 23)).astype(jnp.int32)
    body = functools.partial(_rank_body, L=L, len_keep=len_keep)
    rank3, mask3 = pl.pallas_call(
        body,
        grid=(N, L // _IC),
        in_specs=[
            pl.BlockSpec((1, 1, L), lambda r, ic: (r, 0, 0)),
            pl.BlockSpec((1, _IC, 1), lambda r, ic: (r, ic, 0)),
        ],
        out_specs=[
            pl.BlockSpec((1, _IC, 1), lambda r, ic: (r, ic, 0)),
            pl.BlockSpec((1, _IC, 1), lambda r, ic: (r, ic, 0)),
        ],
        out_shape=[
            jax.ShapeDtypeStruct((N, L, 1), jnp.int32),
            jax.ShapeDtypeStruct((N, L, 1), jnp.float32),
        ],
        interpret=interpret,
    )(keys.reshape(N, 1, L), keys.reshape(N, L, 1))
    return rank3.reshape(N, L), mask3.reshape(N, L)


_CR = 4


def _make_sc_gather(N, F, L, D, len_keep):
    n_rows = N * F * D
    workers = 32
    rows_per_w = n_rows // workers
    w_per_n = workers // N
    chunks = rows_per_w // _CR
    mesh = plsc.VectorSubcoreMesh(core_axis_name="c", subcore_axis_name="s")

    @functools.partial(
        pl.kernel,
        mesh=mesh,
        out_type=jax.ShapeDtypeStruct((n_rows, len_keep), jnp.float32),
        scratch_types=(
            [pltpu.VMEM((L,), jnp.int32), pltpu.VMEM((len_keep,), jnp.int32)]
            + [pltpu.VMEM((L,), jnp.float32) for _ in range(2 * _CR)]
            + [pltpu.VMEM((len_keep,), jnp.float32) for _ in range(2 * _CR)]
            + [pltpu.SemaphoreType.DMA for _ in range(4)]
        ),
        compiler_params=pltpu.CompilerParams(needs_layout_passes=False),
    )
    def sc_gather(rank_hbm, xt_hbm, yt_hbm, rank_v, keep_v, *rest):
        ins = (rest[:_CR], rest[_CR:2 * _CR])
        outs = (rest[2 * _CR:3 * _CR], rest[3 * _CR:4 * _CR])
        sem_i = rest[4 * _CR:4 * _CR + 2]
        sem_o = rest[4 * _CR + 2:4 * _CR + 4]
        c = lax.axis_index("c")
        s = lax.axis_index("s")
        w = s * 2 + c
        n = w // w_per_n
        row0 = w * rows_per_w

        pltpu.sync_copy(rank_hbm.at[n], rank_v)

        def inv_body(i, carry):
            rk = rank_v[pl.ds(i * 16, 16)]
            vals = i * 16 + lax.iota(jnp.int32, 16)
            keepm = rk < len_keep
            idxc = jnp.where(keepm, rk, len_keep - 1)
            plsc.store_scatter(keep_v, [idxc], vals, mask=keepm)
            return carry

        lax.fori_loop(0, L // 16, inv_body, 0)

        def start_ins(ch, b):
            rb = row0 + ch * _CR
            for rr in range(_CR):
                pltpu.async_copy(xt_hbm.at[rb + rr], ins[b][rr], sem_i[b])

        def wait_ins(ch, b):
            rb = row0 + ch * _CR
            for rr in range(_CR):
                pltpu.make_async_copy(
                    xt_hbm.at[rb + rr], ins[b][rr], sem_i[b]).wait()

        def wait_outs(ch, b):
            rb = row0 + ch * _CR
            for rr in range(_CR):
                pltpu.make_async_copy(
                    outs[b][rr], yt_hbm.at[rb + rr], sem_o[b]).wait()

        def compute_and_flush(ch, b):
            def q_body(q, carry2):
                idx = keep_v[pl.ds(q * 16, 16)]
                for rr in range(_CR):
                    outs[b][rr][pl.ds(q * 16, 16)] = plsc.load_gather(
                        ins[b][rr], [idx])
                return carry2

            lax.fori_loop(0, len_keep // 16, q_body, 0)
            rb = row0 + ch * _CR
            for rr in range(_CR):
                pltpu.async_copy(outs[b][rr], yt_hbm.at[rb + rr], sem_o[b])

        start_ins(0, 0)

        def pair_body(u, carry):
            ch_a = 2 * u
            ch_b = 2 * u + 1
            start_ins(ch_b, 1)
            wait_ins(ch_a, 0)

            @pl.when(u > 0)
            def _():
                wait_outs(ch_a - 2, 0)

            compute_and_flush(ch_a, 0)

            @pl.when(u + 1 < chunks // 2)
            def _():
                start_ins(ch_a + 2, 0)

            wait_ins(ch_b, 1)

            @pl.when(u > 0)
            def _():
                wait_outs(ch_b - 2, 1)

            compute_and_flush(ch_b, 1)
            return carry

        lax.fori_loop(0, chunks // 2, pair_body, 0)
        wait_outs(chunks - 2, 0)
        wait_outs(chunks - 1, 1)

    return sc_gather


def kernel(x):
    N, F, L, D = x.shape
    len_keep = int(L * (1 - _MASK_RATIO))
    noise = jax.random.uniform(jax.random.key(42), (N, L), dtype=x.dtype)
    rank, mask_row = _compute_rank(noise)
    sc_gather = _make_sc_gather(N, F, L, D, len_keep)
    xt = jnp.transpose(x, (0, 1, 3, 2)).reshape(N * F * D, L)
    yt = sc_gather(rank, xt)
    x_masked = jnp.transpose(yt.reshape(N, F, D, len_keep), (0, 1, 3, 2))
    mask = jnp.broadcast_to(mask_row[:, None, :], (N, F, L))
    ids_restore = jnp.broadcast_to(rank[:, None, :], (N, F, L))
    return (x_masked, mask, ids_restore)
